# baseline (speedup 1.0000x reference)
import jax
import jax.numpy as jnp
from jax import lax
from jax.experimental import pallas as pl
from jax.experimental.pallas import tpu as pltpu
from jax.experimental.pallas import tpu_sc as plsc

B = 16384
NF = 100
NC = 26
CARD = 1000
D = 128
TOK = NF + NC

NUM_CORES = 2
NUM_SUBCORES = 16
NW = NUM_CORES * NUM_SUBCORES
ROWS_PER_W = B * NC // NW
CHUNK_R = 128
N_CHUNKS = ROWS_PER_W // CHUNK_R
CAT_BASE = NF * B


def _sc_body(table_hbm, idx_hbm, out_hbm,
             idx_v, rows0, rows1, sg0, sg1, sf0, sf1):
    cid = lax.axis_index("c")
    sid = lax.axis_index("s")
    wid = sid * NUM_CORES + cid
    base_r = wid * ROWS_PER_W

    pltpu.sync_copy(idx_hbm.at[wid], idx_v)

    bufs = (rows0, rows1)
    gsems = (sg0, sg1)
    fsems = (sf0, sf1)

    def out_at(g):
        return out_hbm.at[pl.ds(CAT_BASE + base_r + g * CHUNK_R, CHUNK_R)]

    pltpu.async_copy(table_hbm.at[idx_v.at[0]], rows0, sg0)

    def pair(p, carry):
        for b in range(2):
            g = 2 * p + b
            buf, gs = bufs[b], gsems[b]
            obuf, ofs = bufs[1 - b], fsems[1 - b]
            pltpu.make_async_copy(table_hbm.at[idx_v.at[g]], buf, gs).wait()

            @pl.when(g >= 1)
            def _():
                pltpu.make_async_copy(obuf, out_at(0), ofs).wait()

            @pl.when(g + 1 < N_CHUNKS)
            def _():
                pltpu.async_copy(table_hbm.at[idx_v.at[g + 1]], obuf, gsems[1 - b])

            pltpu.async_copy(buf, out_at(g), fsems[b])
        return carry

    lax.fori_loop(0, N_CHUNKS // 2, pair, 0)
    pltpu.make_async_copy(rows1, out_at(N_CHUNKS - 1), sf1).wait()


def _sc_gather(emb_table, gidx):
    mesh = plsc.VectorSubcoreMesh(core_axis_name="c", subcore_axis_name="s")
    return pl.kernel(
        _sc_body,
        out_type=jax.ShapeDtypeStruct((TOK * B, D), jnp.float32),
        mesh=mesh,
        scratch_types=[
            pltpu.VMEM((N_CHUNKS, CHUNK_R), jnp.int32),
            pltpu.VMEM((CHUNK_R, D), jnp.float32),
            pltpu.VMEM((CHUNK_R, D), jnp.float32),
            pltpu.SemaphoreType.DMA,
            pltpu.SemaphoreType.DMA,
            pltpu.SemaphoreType.DMA,
            pltpu.SemaphoreType.DMA,
        ],
    )(emb_table, gidx)


def _fold_bias_body(t_ref, b_ref, o_ref):
    o_ref[...] = t_ref[...] + b_ref[pl.ds(pl.program_id(0), 1)]


def _fold_bias(emb_table, cat_bias):
    return pl.pallas_call(
        _fold_bias_body,
        grid=(NC,),
        in_specs=[
            pl.BlockSpec((CARD, D), lambda i: (i, 0)),
            pl.BlockSpec((NC, D), lambda i: (0, 0)),
        ],
        out_specs=pl.BlockSpec((CARD, D), lambda i: (i, 0)),
        out_shape=jax.ShapeDtypeStruct((NC * CARD, D), jnp.float32),
    )(emb_table, cat_bias)


BB = 128


def _tc_num_body(x_ref, w_ref, b_ref, prev_ref, out_ref):
    del prev_ref
    for f in range(NF):
        out_ref[f] = x_ref[:, f : f + 1] * w_ref[f : f + 1, :] + b_ref[f : f + 1, :]


def _tc_num(x_num, num_weight, num_bias, prev):
    return pl.pallas_call(
        _tc_num_body,
        grid=(B // BB,),
        in_specs=[
            pl.BlockSpec((BB, NF), lambda i: (i, 0)),
            pl.BlockSpec((NF, D), lambda i: (0, 0)),
            pl.BlockSpec((NF, D), lambda i: (0, 0)),
            pl.BlockSpec(memory_space=pl.ANY),
        ],
        out_specs=pl.BlockSpec((NF, BB, D), lambda i: (0, i, 0)),
        out_shape=jax.ShapeDtypeStruct((TOK, B, D), jnp.float32),
        input_output_aliases={3: 0},
    )(x_num, num_weight, num_bias, prev)


def kernel(x_num, x_cat, num_weight, num_bias, emb_table, cat_bias):
    offsets = jnp.arange(NC, dtype=jnp.int32) * CARD
    gidx = (x_cat.astype(jnp.int32).T + offsets[:, None]).reshape(
        NW, N_CHUNKS, CHUNK_R
    )
    table_b = _fold_bias(emb_table, cat_bias)
    rows = _sc_gather(table_b, gidx)
    out_t = _tc_num(x_num, num_weight, num_bias, rows.reshape(TOK, B, D))
    return out_t.transpose(1, 0, 2)


# speedup vs baseline: 1.1050x; 1.1050x over previous
import jax
import jax.numpy as jnp
from jax import lax
from jax.experimental import pallas as pl
from jax.experimental.pallas import tpu as pltpu
from jax.experimental.pallas import tpu_sc as plsc

B = 16384
NF = 100
NC = 26
CARD = 1000
D = 128
TOK = NF + NC

NUM_CORES = 2
NUM_SUBCORES = 16
NW = NUM_CORES * NUM_SUBCORES
ROWS_PER_W = B * NC // NW
CHUNK_R = 128
N_CHUNKS = ROWS_PER_W // CHUNK_R
CAT_BASE = NF * B


def _sc_body(table_hbm, idx_hbm, bias_hbm, out_hbm,
             idx_v, bias_v, rows0, rows1, sem0, sem1):
    cid = lax.axis_index("c")
    sid = lax.axis_index("s")
    wid = sid * NUM_CORES + cid
    base_r = wid * ROWS_PER_W

    pltpu.sync_copy(idx_hbm.at[wid], idx_v)
    pltpu.sync_copy(bias_hbm, bias_v)

    def bias_add_and_flush(rows, g):
        f = (base_r + g * CHUNK_R) // B
        bvs = [bias_v[f, pl.ds(v * 16, 16)] for v in range(8)]

        def radd(r, c):
            for v in range(8):
                sl = pl.ds(v * 16, 16)
                rows[r, sl] = rows[r, sl] + bvs[v]
            return c

        lax.fori_loop(0, CHUNK_R, radd, 0)
        pltpu.sync_copy(
            rows, out_hbm.at[pl.ds(CAT_BASE + base_r + g * CHUNK_R, CHUNK_R)]
        )

    pltpu.async_copy(table_hbm.at[idx_v.at[0]], rows0, sem0)

    def pair(p, carry):
        g0 = 2 * p
        g1 = g0 + 1
        pltpu.make_async_copy(table_hbm.at[idx_v.at[g0]], rows0, sem0).wait()
        pltpu.async_copy(table_hbm.at[idx_v.at[g1]], rows1, sem1)
        bias_add_and_flush(rows0, g0)
        pltpu.make_async_copy(table_hbm.at[idx_v.at[g1]], rows1, sem1).wait()

        @pl.when(g1 + 1 < N_CHUNKS)
        def _():
            pltpu.async_copy(table_hbm.at[idx_v.at[g1 + 1]], rows0, sem0)

        bias_add_and_flush(rows1, g1)
        return carry

    lax.fori_loop(0, N_CHUNKS // 2, pair, 0)


def _sc_gather(emb_table, gidx, cat_bias):
    mesh = plsc.VectorSubcoreMesh(core_axis_name="c", subcore_axis_name="s")
    return pl.kernel(
        _sc_body,
        out_type=jax.ShapeDtypeStruct((TOK * B, D), jnp.float32),
        mesh=mesh,
        scratch_types=[
            pltpu.VMEM((N_CHUNKS, CHUNK_R), jnp.int32),
            pltpu.VMEM((NC, D), jnp.float32),
            pltpu.VMEM((CHUNK_R, D), jnp.float32),
            pltpu.VMEM((CHUNK_R, D), jnp.float32),
            pltpu.SemaphoreType.DMA,
            pltpu.SemaphoreType.DMA,
        ],
    )(emb_table, gidx, cat_bias)


BB = 256


def _tc_num_body(x_ref, w_ref, b_ref, prev_ref, out_ref):
    del prev_ref
    for f in range(NF):
        out_ref[f] = x_ref[:, f : f + 1] * w_ref[f : f + 1, :] + b_ref[f : f + 1, :]


def _tc_num(x_num, num_weight, num_bias, prev):
    return pl.pallas_call(
        _tc_num_body,
        grid=(B // BB,),
        in_specs=[
            pl.BlockSpec((BB, NF), lambda i: (i, 0)),
            pl.BlockSpec((NF, D), lambda i: (0, 0)),
            pl.BlockSpec((NF, D), lambda i: (0, 0)),
            pl.BlockSpec(memory_space=pl.ANY),
        ],
        out_specs=pl.BlockSpec((NF, BB, D), lambda i: (0, i, 0)),
        out_shape=jax.ShapeDtypeStruct((TOK, B, D), jnp.float32),
        input_output_aliases={3: 0},
    )(x_num, num_weight, num_bias, prev)


def kernel(x_num, x_cat, num_weight, num_bias, emb_table, cat_bias):
    offsets = jnp.arange(NC, dtype=jnp.int32) * CARD
    gidx = (x_cat.astype(jnp.int32).T + offsets[:, None]).reshape(
        NW, N_CHUNKS, CHUNK_R
    )
    rows = _sc_gather(emb_table, gidx, cat_bias)
    out_t = _tc_num(x_num, num_weight, num_bias, rows.reshape(TOK, B, D))
    return out_t.transpose(1, 0, 2)


# speedup vs baseline: 1.1760x; 1.0643x over previous
import jax
import jax.numpy as jnp
from jax import lax
from jax.experimental import pallas as pl
from jax.experimental.pallas import tpu as pltpu
from jax.experimental.pallas import tpu_sc as plsc

B = 16384
NF = 100
NC = 26
CARD = 1000
D = 128
TOK = NF + NC

NUM_CORES = 2
NUM_SUBCORES = 16
NW = NUM_CORES * NUM_SUBCORES
ROWS_PER_W = B * NC // NW
CHUNK_R = 128
N_CHUNKS = ROWS_PER_W // CHUNK_R
CAT_BASE = NF * B


def _sc_body(table_hbm, idx_hbm, bias_hbm, out_hbm,
             idx_v, bias_v, rows0, rows1, sem0, sem1):
    cid = lax.axis_index("c")
    sid = lax.axis_index("s")
    wid = sid * NUM_CORES + cid
    base_r = wid * ROWS_PER_W

    pltpu.sync_copy(idx_hbm.at[wid], idx_v)
    pltpu.sync_copy(bias_hbm, bias_v)

    def bias_add_and_flush(rows, g):
        f = (base_r + g * CHUNK_R) // B
        bvs = [bias_v[f, pl.ds(v * 16, 16)] for v in range(8)]

        def radd(r, c):
            for v in range(8):
                sl = pl.ds(v * 16, 16)
                rows[r, sl] = rows[r, sl] + bvs[v]
            return c

        lax.fori_loop(0, CHUNK_R, radd, 0)
        pltpu.sync_copy(
            rows, out_hbm.at[pl.ds(CAT_BASE + base_r + g * CHUNK_R, CHUNK_R)]
        )

    pltpu.async_copy(table_hbm.at[idx_v.at[0]], rows0, sem0)

    def pair(p, carry):
        g0 = 2 * p
        g1 = g0 + 1
        pltpu.make_async_copy(table_hbm.at[idx_v.at[g0]], rows0, sem0).wait()
        pltpu.async_copy(table_hbm.at[idx_v.at[g1]], rows1, sem1)
        bias_add_and_flush(rows0, g0)
        pltpu.make_async_copy(table_hbm.at[idx_v.at[g1]], rows1, sem1).wait()

        @pl.when(g1 + 1 < N_CHUNKS)
        def _():
            pltpu.async_copy(table_hbm.at[idx_v.at[g1 + 1]], rows0, sem0)

        bias_add_and_flush(rows1, g1)
        return carry

    lax.fori_loop(0, N_CHUNKS // 2, pair, 0)


def _sc_gather(emb_table, gidx, cat_bias):
    mesh = plsc.VectorSubcoreMesh(core_axis_name="c", subcore_axis_name="s")
    return pl.kernel(
        _sc_body,
        out_type=jax.ShapeDtypeStruct((TOK * B, D), jnp.float32),
        mesh=mesh,
        scratch_types=[
            pltpu.VMEM((N_CHUNKS, CHUNK_R), jnp.int32),
            pltpu.VMEM((NC, D), jnp.float32),
            pltpu.VMEM((CHUNK_R, D), jnp.float32),
            pltpu.VMEM((CHUNK_R, D), jnp.float32),
            pltpu.SemaphoreType.DMA,
            pltpu.SemaphoreType.DMA,
        ],
    )(emb_table, gidx, cat_bias)


BB = 512


def _tc_num_body(x_ref, w_ref, b_ref, prev_ref, out_ref):
    del prev_ref
    for f in range(NF):
        out_ref[f] = x_ref[:, f : f + 1] * w_ref[f : f + 1, :] + b_ref[f : f + 1, :]


def _tc_num(x_num, num_weight, num_bias, prev):
    return pl.pallas_call(
        _tc_num_body,
        grid=(B // BB,),
        in_specs=[
            pl.BlockSpec((BB, NF), lambda i: (i, 0)),
            pl.BlockSpec((NF, D), lambda i: (0, 0)),
            pl.BlockSpec((NF, D), lambda i: (0, 0)),
            pl.BlockSpec(memory_space=pl.ANY),
        ],
        out_specs=pl.BlockSpec((NF, BB, D), lambda i: (0, i, 0)),
        out_shape=jax.ShapeDtypeStruct((TOK, B, D), jnp.float32),
        input_output_aliases={3: 0},
    )(x_num, num_weight, num_bias, prev)


def kernel(x_num, x_cat, num_weight, num_bias, emb_table, cat_bias):
    offsets = jnp.arange(NC, dtype=jnp.int32) * CARD
    gidx = (x_cat.astype(jnp.int32).T + offsets[:, None]).reshape(
        NW, N_CHUNKS, CHUNK_R
    )
    rows = _sc_gather(emb_table, gidx, cat_bias)
    out_t = _tc_num(x_num, num_weight, num_bias, rows.reshape(TOK, B, D))
    return out_t.transpose(1, 0, 2)
